# Initial kernel scaffold; baseline (speedup 1.0000x reference)
import jax
import jax.numpy as jnp
from jax.experimental import pallas as pl


def kernel(x, edge_index, batch, W1, b1, W2, att_src, att_dst, b2, fc1_w, fc1_b, fc2_w, fc2_b):
    raise NotImplementedError("write your pallas kernel here")


# speedup vs baseline: 2.0313x; 2.0313x over previous
import functools
import jax
import jax.numpy as jnp
from jax.experimental import pallas as pl

TE = 256
G_OUT = 64
_F32 = jnp.float32


def _deg_kernel(dstb, deg):
    i = pl.program_id(0)
    npad = deg.shape[0]
    d = dstb[0, 0, :].reshape(TE, 1)
    oh = (d == jax.lax.broadcasted_iota(jnp.int32, (TE, npad), 1)).astype(_F32)
    upd = jax.lax.dot_general(oh, jnp.ones((TE, 8), _F32),
                              (((0,), (0,)), ((), ())),
                              preferred_element_type=_F32)

    @pl.when(i == 0)
    def _():
        deg[...] = jnp.zeros_like(deg)

    deg[...] += upd


def _h1s_kernel(x, w1, deg, h1s):
    d = deg[:, 0:1]
    dinv = jnp.where(d > 0, jax.lax.rsqrt(d), 0.0)
    h = jnp.dot(x[...], w1[...], preferred_element_type=_F32)
    h1s[...] = h * dinv


def _gcn_kernel(srcb, dstb, h1s, deg, b1, x1, *, nsteps):
    i = pl.program_id(0)
    npad = h1s.shape[0]
    s = srcb[0, 0, :].reshape(TE, 1)
    d = dstb[0, 0, :].reshape(TE, 1)
    iot = jax.lax.broadcasted_iota(jnp.int32, (TE, npad), 1)
    ohs = (s == iot).astype(_F32)
    ohd = (d == iot).astype(_F32)
    msg = jnp.dot(ohs, h1s[...], preferred_element_type=_F32)
    upd = jax.lax.dot_general(ohd, msg, (((0,), (0,)), ((), ())),
                              preferred_element_type=_F32)

    @pl.when(i == 0)
    def _():
        x1[...] = jnp.zeros_like(x1)

    x1[...] += upd

    @pl.when(i == nsteps - 1)
    def _():
        dg = deg[:, 0:1]
        dinv = jnp.where(dg > 0, jax.lax.rsqrt(dg), 0.0)
        x1[...] = jnp.maximum(x1[...] * dinv + b1[0:1, :], 0.0)


def _node2_kernel(x1, w2, bs, bd, h2t, ast, adt):
    h2 = jnp.dot(x1[...], w2[...], preferred_element_type=_F32)
    h2t[...] = h2
    ast[...] = jnp.dot(h2, bs[...], preferred_element_type=_F32)
    adt[...] = jnp.dot(h2, bd[...], preferred_element_type=_F32)


def _gat_kernel(srcb, dstb, ast, adt, h2t, b2, erep, den, x2, *, nsteps):
    i = pl.program_id(0)
    npad = h2t.shape[0]
    s = srcb[0, 0, :].reshape(TE, 1)
    d = dstb[0, 0, :].reshape(TE, 1)
    iot = jax.lax.broadcasted_iota(jnp.int32, (TE, npad), 1)
    ohs = (s == iot).astype(_F32)
    ohd = (d == iot).astype(_F32)
    es = jnp.dot(ohs, ast[...], preferred_element_type=_F32)
    ed = jnp.dot(ohd, adt[...], preferred_element_type=_F32)
    e = es + ed
    e = jnp.where(e >= 0, e, 0.2 * e)
    ex = jnp.exp(e)
    h2s = jnp.dot(ohs, h2t[...], preferred_element_type=_F32)
    ex128 = jnp.dot(ex, erep[...], preferred_element_type=_F32)
    wm = ex128 * h2s

    @pl.when(i == 0)
    def _():
        den[...] = jnp.zeros_like(den)
        x2[...] = jnp.zeros_like(x2)

    den[...] += jax.lax.dot_general(ohd, ex, (((0,), (0,)), ((), ())),
                                    preferred_element_type=_F32)
    x2[...] += jax.lax.dot_general(ohd, wm, (((0,), (0,)), ((), ())),
                                   preferred_element_type=_F32)

    @pl.when(i == nsteps - 1)
    def _():
        d128 = jnp.dot(den[...], erep[...], preferred_element_type=_F32)
        x2[...] = jnp.maximum(x2[...] / (d128 + 1e-16) + b2[0:1, :], 0.0)


def _pool_kernel(batchb, x1, x2, f1a, f1b, f1bias, f2w, f2bias, out):
    npad = x1.shape[0]
    b = batchb[0, 0, :].reshape(npad, 1)
    oh = (b == jax.lax.broadcasted_iota(jnp.int32, (npad, G_OUT), 1)).astype(_F32)
    dn = (((0,), (0,)), ((), ()))
    s1 = jax.lax.dot_general(oh, x1[...], dn, preferred_element_type=_F32)
    s2 = jax.lax.dot_general(oh, x2[...], dn, preferred_element_type=_F32)
    cnt = jax.lax.dot_general(oh, jnp.ones((npad, 8), _F32), dn,
                              preferred_element_type=_F32)
    c = jnp.maximum(cnt[:, 0:1], 1.0)
    p1 = s1 / c
    p2 = s2 / c
    hfc = jnp.dot(p1, f1a[...], preferred_element_type=_F32)
    hfc += jnp.dot(p2, f1b[...], preferred_element_type=_F32)
    hfc = jnp.maximum(hfc + f1bias[0:1, :], 0.0)
    out[...] = jnp.dot(hfc, f2w[...], preferred_element_type=_F32) + f2bias[0:1, :]


def kernel(x, edge_index, batch, W1, b1, W2, att_src, att_dst, b2, fc1_w, fc1_b, fc2_w, fc2_b):
    n, f_in = x.shape
    hid = W1.shape[1]
    h = att_src.shape[0]
    hh = W2.shape[1]

    npad = ((n + 255) // 256) * 256
    e_real = edge_index.shape[1] + n
    nblk = (e_real + TE - 1) // TE
    e_pad = nblk * TE

    loop = jnp.arange(n, dtype=jnp.int32)
    src = jnp.concatenate([edge_index[0], loop])
    dst = jnp.concatenate([edge_index[1], loop])
    fill = jnp.full((e_pad - e_real,), n, jnp.int32)
    srcb = jnp.concatenate([src, fill]).reshape(nblk, 1, TE)
    dstb = jnp.concatenate([dst, fill]).reshape(nblk, 1, TE)

    xp = jnp.zeros((npad, f_in), _F32).at[:n].set(x)
    batchp = jnp.concatenate([batch, jnp.full((npad - n,), G_OUT, jnp.int32)])
    batchb = batchp.reshape(1, 1, npad)

    b1r = jnp.broadcast_to(b1.reshape(1, hid), (8, hid))
    b2r = jnp.broadcast_to(b2.reshape(1, hh), (8, hh))
    f1br = jnp.broadcast_to(fc1_b.reshape(1, hid), (8, hid))
    f2br = jnp.broadcast_to(fc2_b.reshape(1, 1), (8, 1))
    eye = jnp.eye(h, dtype=_F32)
    bs = (eye[:, None, :] * att_src[:, :, None]).reshape(hh, h)
    bd = (eye[:, None, :] * att_dst[:, :, None]).reshape(hh, h)
    erep = jnp.repeat(eye, hid, axis=1)
    f1a = fc1_w[:hid]
    f1b = fc1_w[hid:]

    eblk = pl.BlockSpec((1, 1, TE), lambda i: (i, 0, 0))
    full = lambda shp: pl.BlockSpec(shp, lambda i: (0,) * len(shp))

    deg = pl.pallas_call(
        _deg_kernel,
        grid=(nblk,),
        in_specs=[eblk],
        out_specs=full((npad, 8)),
        out_shape=jax.ShapeDtypeStruct((npad, 8), _F32),
    )(dstb)

    h1s = pl.pallas_call(
        _h1s_kernel,
        out_shape=jax.ShapeDtypeStruct((npad, hid), _F32),
    )(xp, W1, deg)

    x1 = pl.pallas_call(
        functools.partial(_gcn_kernel, nsteps=nblk),
        grid=(nblk,),
        in_specs=[eblk, eblk, full((npad, hid)), full((npad, 8)), full((8, hid))],
        out_specs=full((npad, hid)),
        out_shape=jax.ShapeDtypeStruct((npad, hid), _F32),
    )(srcb, dstb, h1s, deg, b1r)

    h2t, ast, adt = pl.pallas_call(
        _node2_kernel,
        out_shape=[jax.ShapeDtypeStruct((npad, hh), _F32),
                   jax.ShapeDtypeStruct((npad, h), _F32),
                   jax.ShapeDtypeStruct((npad, h), _F32)],
    )(x1, W2, bs, bd)

    _, x2 = pl.pallas_call(
        functools.partial(_gat_kernel, nsteps=nblk),
        grid=(nblk,),
        in_specs=[eblk, eblk, full((npad, h)), full((npad, h)),
                  full((npad, hh)), full((8, hh)), full((h, hh))],
        out_specs=[full((npad, h)), full((npad, hh))],
        out_shape=[jax.ShapeDtypeStruct((npad, h), _F32),
                   jax.ShapeDtypeStruct((npad, hh), _F32)],
    )(srcb, dstb, ast, adt, h2t, b2r, erep)

    out = pl.pallas_call(
        _pool_kernel,
        out_shape=jax.ShapeDtypeStruct((G_OUT, 1), _F32),
    )(batchb, x1, x2, f1a, f1b, f1br, fc2_w, f2br)

    return out.reshape(G_OUT)
